# baseline (speedup 1.0000x reference)
import jax
import jax.numpy as jnp
from jax import lax
from jax.experimental import pallas as pl
from jax.experimental.pallas import tpu as pltpu

_B, _S, _T, _M = 16, 512, 2048, 80
_LW = 640
_GR = _LW // _M
_ROWS = _B * _T * _M // _LW
_BR = 512
_NSTEP = _ROWS // _BR


def _body(melt_ref, melp_ref, pn_ref, v_ref, oh_ref, pp_ref, pt_ref, ep_ref,
          et_ref, lp_ref, dt_ref, sm_ref, out_ref):
    i = pl.program_id(0)

    mexp = lax.dot_general(
        v_ref[...], oh_ref[...], (((1,), (0,)), ((), ())),
        precision=lax.Precision.HIGHEST, preferred_element_type=jnp.float32)
    t = melt_ref[...]
    s_mel = jnp.sum(jnp.abs(melp_ref[...] - t) * mexp)
    s_pn = jnp.sum(jnp.abs(pn_ref[...] - t) * mexp)
    s_cnt = jnp.sum(v_ref[...])

    @pl.when(i == 0)
    def _init():
        srcv = sm_ref[...]
        dp = pp_ref[...] - pt_ref[...]
        de = ep_ref[...] - et_ref[...]
        dd = lp_ref[...] - jnp.log(dt_ref[...] + 1.0)
        out_ref[0] = s_mel
        out_ref[1] = s_pn
        out_ref[2] = s_cnt
        out_ref[3] = jnp.sum(dp * dp * srcv)
        out_ref[4] = jnp.sum(de * de * srcv)
        out_ref[5] = jnp.sum(dd * dd * srcv)
        out_ref[6] = jnp.sum(srcv)

    @pl.when(i != 0)
    def _acc():
        out_ref[0] += s_mel
        out_ref[1] += s_pn
        out_ref[2] += s_cnt


def _losses(mel_t, mel_p, pn_p, valid_f, onehot, pitch_p, pitch_t, energy_p,
            energy_t, logdur_p, dur_f, src_valid):
    big = pl.BlockSpec((_BR, _LW), lambda i: (i, 0))
    vmask = pl.BlockSpec((_BR, _GR), lambda i: (i, 0))
    ohs = pl.BlockSpec((_GR, _LW), lambda i: (0, 0))
    small = pl.BlockSpec((_B, _S), lambda i: (0, 0))
    return pl.pallas_call(
        _body,
        grid=(_NSTEP,),
        in_specs=[big, big, big, vmask, ohs, small, small, small, small,
                  small, small, small],
        out_specs=pl.BlockSpec(memory_space=pltpu.SMEM),
        out_shape=jax.ShapeDtypeStruct((7,), jnp.float32),
    )(mel_t, mel_p, pn_p, valid_f, onehot, pitch_p, pitch_t, energy_p,
      energy_t, logdur_p, dur_f, src_valid)


def kernel(mel_targets, pitch_targets, energy_targets, duration_targets,
           mel_predictions, postnet_mel_predictions, pitch_predictions,
           energy_predictions, log_duration_predictions, src_masks,
           mel_masks):
    valid_f = (~mel_masks).astype(jnp.float32).reshape(_ROWS, _GR)
    src_valid = (~src_masks).astype(jnp.float32)
    dur_f = duration_targets.astype(jnp.float32)
    onehot = (jnp.arange(_LW, dtype=jnp.int32)[None, :] // _M
              == jnp.arange(_GR, dtype=jnp.int32)[:, None]
              ).astype(jnp.float32)

    sums = _losses(mel_targets.reshape(_ROWS, _LW),
                   mel_predictions.reshape(_ROWS, _LW),
                   postnet_mel_predictions.reshape(_ROWS, _LW),
                   valid_f, onehot, pitch_predictions, pitch_targets,
                   energy_predictions, energy_targets,
                   log_duration_predictions, dur_f, src_valid)

    mel_den = jnp.maximum(sums[2] * _M, 1.0)
    src_den = jnp.maximum(sums[6], 1.0)
    mel_loss = sums[0] / mel_den
    postnet_mel_loss = sums[1] / mel_den
    pitch_loss = sums[3] / src_den
    energy_loss = sums[4] / src_den
    duration_loss = sums[5] / src_den
    total_loss = (mel_loss + postnet_mel_loss + duration_loss + pitch_loss
                  + energy_loss)
    return (total_loss, mel_loss, postnet_mel_loss, pitch_loss, energy_loss,
            duration_loss)
